# Initial kernel scaffold; baseline (speedup 1.0000x reference)
import jax
import jax.numpy as jnp
from jax import lax
from jax.experimental import pallas as pl
from jax.experimental.pallas import tpu as pltpu
from jax.experimental.pallas import tpu_sc as plsc

B, K_SLOTS, N, DIM, NUM_ENTRIES = 16, 64, 1024, 768, 8192
ROWS = B * K_SLOTS

_info = plsc.get_sparse_core_info()
_NC, _NS = _info.num_cores, _info.num_subcores
_NW = _NC * _NS
_ROWS_PER_W = ROWS // _NW


def _sc_gather_body(idx_hbm, table_hbm, out_hbm, idx_v, rows_v, sem):
    wid = lax.axis_index("s") * _NC + lax.axis_index("c")
    base = wid * _ROWS_PER_W
    pltpu.sync_copy(idx_hbm.at[pl.ds(base, _ROWS_PER_W)], idx_v)
    pltpu.async_copy(table_hbm.at[idx_v], rows_v, sem).wait()
    pltpu.sync_copy(rows_v, out_hbm.at[pl.ds(base, _ROWS_PER_W)])


_sc_gather = pl.kernel(
    _sc_gather_body,
    out_type=jax.ShapeDtypeStruct((ROWS, DIM), jnp.float32),
    mesh=plsc.VectorSubcoreMesh(core_axis_name="c", subcore_axis_name="s"),
    scratch_types=[
        pltpu.VMEM((_ROWS_PER_W,), jnp.int32),
        pltpu.VMEM((_ROWS_PER_W, DIM), jnp.float32),
        pltpu.SemaphoreType.DMA,
    ],
)


def _mm_body(attn_ref, rows_ref, out_ref):
    a = attn_ref[0]
    r = rows_ref[0]
    out_ref[0] = lax.dot_general(
        a, r, (((0,), (0,)), ((), ())), preferred_element_type=jnp.float32
    )


@jax.jit
def kernel(indices, attn_map, prompt):
    idx_flat = indices.reshape(ROWS).astype(jnp.int32)
    rows = _sc_gather(idx_flat, prompt)
    rows = rows.reshape(B, K_SLOTS, DIM)
    out = pl.pallas_call(
        _mm_body,
        grid=(B,),
        in_specs=[
            pl.BlockSpec((1, K_SLOTS, N), lambda b: (b, 0, 0)),
            pl.BlockSpec((1, K_SLOTS, DIM), lambda b: (b, 0, 0)),
        ],
        out_specs=pl.BlockSpec((1, N, DIM), lambda b: (b, 0, 0)),
        out_shape=jax.ShapeDtypeStruct((B, N, DIM), jnp.float32),
    )(attn_map, rows)
    return out


# speedup vs baseline: 1.9230x; 1.9230x over previous
import jax
import jax.numpy as jnp
from jax import lax
from jax.experimental import pallas as pl
from jax.experimental.pallas import tpu as pltpu
from jax.experimental.pallas import tpu_sc as plsc

B, K_SLOTS, N, DIM, NUM_ENTRIES = 16, 64, 1024, 768, 8192
ROWS = B * K_SLOTS

_NC, _NS = 2, 16
_NW = _NC * _NS
_ROWS_PER_W = ROWS // _NW


def _sc_gather_body(idx_hbm, table_hbm, out_hbm, idx_v, rows_v, sem):
    wid = lax.axis_index("s") * _NC + lax.axis_index("c")
    base = wid * _ROWS_PER_W
    pltpu.sync_copy(idx_hbm.at[pl.ds(base, _ROWS_PER_W)], idx_v)
    pltpu.async_copy(table_hbm.at[idx_v], rows_v, sem).wait()
    pltpu.sync_copy(rows_v, out_hbm.at[pl.ds(base, _ROWS_PER_W)])


_sc_gather = pl.kernel(
    _sc_gather_body,
    out_type=jax.ShapeDtypeStruct((ROWS, DIM), jnp.float32),
    mesh=plsc.VectorSubcoreMesh(core_axis_name="c", subcore_axis_name="s"),
    scratch_types=[
        pltpu.VMEM((_ROWS_PER_W,), jnp.int32),
        pltpu.VMEM((_ROWS_PER_W, DIM), jnp.float32),
        pltpu.SemaphoreType.DMA,
    ],
)


def _mm_body(attn_ref, rows_ref, out_ref):
    a = attn_ref[0]
    r = rows_ref[0]
    out_ref[0] = lax.dot_general(
        a, r, (((0,), (0,)), ((), ())), preferred_element_type=jnp.float32
    )


@jax.jit
def kernel(indices, attn_map, prompt):
    idx_flat = indices.reshape(ROWS).astype(jnp.int32)
    rows = _sc_gather(idx_flat, prompt)
    rows = rows.reshape(B, K_SLOTS, DIM)
    out = pl.pallas_call(
        _mm_body,
        grid=(B,),
        in_specs=[
            pl.BlockSpec((1, K_SLOTS, N), lambda b: (b, 0, 0)),
            pl.BlockSpec((1, K_SLOTS, DIM), lambda b: (b, 0, 0)),
        ],
        out_specs=pl.BlockSpec((1, N, DIM), lambda b: (b, 0, 0)),
        out_shape=jax.ShapeDtypeStruct((B, N, DIM), jnp.float32),
    )(attn_map, rows)
    return out
